# baseline (speedup 1.0000x reference)
import functools

import jax
import jax.numpy as jnp
import numpy as np
from jax.experimental import pallas as pl
from jax.experimental.pallas import tpu as pltpu
from jax.sharding import Mesh, PartitionSpec as P

N = 4096
BLOCK = 256


def _layer_kernel(sr_ref, sp_ref, hr_ref, hp_ref, hrl_ref, hpl_ref,
                  w_ref, sw_ref, out_r_ref, out_p_ref,
                  vu_ref, vv_ref, self_r_ref, self_p_ref, *, block, k):
    i = pl.program_id(0)

    @pl.when(i == 0)
    def _init():
        hr = hr_ref[...]
        hp = hp_ref[...]
        w0 = w_ref[0]
        w1 = w_ref[1]
        sw = sw_ref[...]
        vu_ref[...] = jnp.concatenate(
            [jnp.dot(hr, w0, preferred_element_type=jnp.float32),
             jnp.dot(hr, w1, preferred_element_type=jnp.float32)],
            axis=1).astype(jnp.bfloat16)
        vv_ref[...] = jnp.concatenate(
            [jnp.dot(hp, w0, preferred_element_type=jnp.float32),
             jnp.dot(hp, w1, preferred_element_type=jnp.float32)],
            axis=1).astype(jnp.bfloat16)
        self_r_ref[...] = jnp.dot(hrl_ref[...], sw,
                                  preferred_element_type=jnp.float32)
        self_p_ref[...] = jnp.dot(hpl_ref[...], sw,
                                  preferred_element_type=jnp.float32)

    vu = vu_ref[...]
    vv = vv_ref[...]
    sr0 = sr_ref[0].astype(jnp.bfloat16)
    sr1 = sr_ref[1].astype(jnp.bfloat16)
    sp0 = sp_ref[0].astype(jnp.bfloat16)
    sp1 = sp_ref[1].astype(jnp.bfloat16)
    rows = pl.ds(i * block, block)
    agg_r = jnp.concatenate(
        [jnp.dot(sr0, vv[:, :k], preferred_element_type=jnp.float32),
         jnp.dot(sr1, vv[:, k:], preferred_element_type=jnp.float32)],
        axis=1)
    agg_p = jnp.concatenate(
        [jnp.dot(sp0, vu[:, :k], preferred_element_type=jnp.float32),
         jnp.dot(sp1, vu[:, k:], preferred_element_type=jnp.float32)],
        axis=1)
    out_r_ref[...] = jnp.maximum(agg_r + self_r_ref[rows, :], 0.0)
    out_p_ref[...] = jnp.maximum(agg_p + self_p_ref[rows, :], 0.0)


def _gcn_layer(S_r, S_p, H_r, H_p, H_r_loc, H_p_loc, W, SW, *, block=BLOCK):
    rows_local = S_r.shape[1]
    d = H_r.shape[1]
    k = W.shape[2]
    nblk = rows_local // block
    kern = functools.partial(_layer_kernel, block=block, k=k)
    full = lambda i: (0, 0)
    sup_spec = pl.BlockSpec((2, block, N), lambda i: (0, i, 0))
    h_shape = jax.ShapeDtypeStruct((rows_local, 2 * k), jnp.float32)
    return pl.pallas_call(
        kern,
        grid_spec=pltpu.PrefetchScalarGridSpec(
            num_scalar_prefetch=0,
            grid=(nblk,),
            in_specs=[
                sup_spec,
                sup_spec,
                pl.BlockSpec((N, d), full),
                pl.BlockSpec((N, d), full),
                pl.BlockSpec((rows_local, d), full),
                pl.BlockSpec((rows_local, d), full),
                pl.BlockSpec((2, d, k), lambda i: (0, 0, 0)),
                pl.BlockSpec((d, 2 * k), full),
            ],
            out_specs=[
                pl.BlockSpec((block, 2 * k), lambda i: (i, 0)),
                pl.BlockSpec((block, 2 * k), lambda i: (i, 0)),
            ],
            scratch_shapes=[
                pltpu.VMEM((N, 2 * k), jnp.bfloat16),
                pltpu.VMEM((N, 2 * k), jnp.bfloat16),
                pltpu.VMEM((rows_local, 2 * k), jnp.float32),
                pltpu.VMEM((rows_local, 2 * k), jnp.float32),
            ],
        ),
        out_shape=[h_shape, h_shape],
        compiler_params=pltpu.CompilerParams(
            dimension_semantics=("arbitrary",),
        ),
    )(S_r, S_p, H_r, H_p, H_r_loc, H_p_loc, W, SW)


def _two_layers(sr, sp, hr, hp, w0, w1, sw0, sw1, *, rows_local, axis=None):
    if axis is not None:
        base = jax.lax.axis_index(axis) * rows_local
        hr_loc = jax.lax.dynamic_slice_in_dim(hr, base, rows_local, 0)
        hp_loc = jax.lax.dynamic_slice_in_dim(hp, base, rows_local, 0)
    else:
        hr_loc, hp_loc = hr, hp
    h1r, h1p = _gcn_layer(sr, sp, hr, hp, hr_loc, hp_loc, w0, sw0)
    if axis is not None:
        h1r_full = jax.lax.all_gather(h1r, axis, axis=0, tiled=True)
        h1p_full = jax.lax.all_gather(h1p, axis, axis=0, tiled=True)
    else:
        h1r_full, h1p_full = h1r, h1p
    out_r, out_p = _gcn_layer(sr, sp, h1r_full, h1p_full, h1r, h1p, w1, sw1)
    if axis is not None:
        out_r = jax.lax.all_gather(out_r, axis, axis=0, tiled=True)
        out_p = jax.lax.all_gather(out_p, axis, axis=0, tiled=True)
    return out_r, out_p


def kernel(RNA_supports, protein_supports, RNA_inputs, protein_inputs,
           W0, W1, SW0, SW1):
    devs = jax.devices()
    ndev = 2 if len(devs) >= 2 and N % (2 * BLOCK) == 0 else 1
    if ndev == 1:
        return _two_layers(RNA_supports, protein_supports,
                           RNA_inputs, protein_inputs,
                           W0, W1, SW0, SW1, rows_local=N)
    mesh = Mesh(np.array(devs[:ndev]), ("x",))
    shard = P(None, "x", None)
    rep = P()
    fn = jax.shard_map(
        functools.partial(_two_layers, rows_local=N // ndev, axis="x"),
        mesh=mesh,
        in_specs=(shard, shard, rep, rep, rep, rep, rep, rep),
        out_specs=(rep, rep),
        check_vma=False,
    )
    return fn(RNA_supports, protein_supports, RNA_inputs, protein_inputs,
              W0, W1, SW0, SW1)


# speedup vs baseline: 3.3806x; 3.3806x over previous
import functools

import jax
import jax.numpy as jnp
from jax.experimental import pallas as pl
from jax.experimental.pallas import tpu as pltpu

N = 4096
CHUNK = 256


def _fused_kernel(sr_ref, sp_ref, h0r_ref, h0p_ref,
                  w0_ref, sw0_ref, w1_ref, sw1_ref,
                  out1r_ref, out1p_ref,
                  vu_ref, vv_ref, self_r_ref, self_p_ref,
                  h1r_ref, h1p_ref):
    l = pl.program_id(0)
    i = pl.program_id(1)
    s = i // 16
    r = i % 16
    rows = pl.ds(r * CHUNK, CHUNK)

    @pl.when(jnp.logical_and(l == 0, i == 0))
    def _init0():
        hr = h0r_ref[...]
        hp = h0p_ref[...]
        sw = sw0_ref[...]
        for j in range(2):
            wj = w0_ref[j]
            vu_ref[j] = jnp.dot(hr, wj,
                                preferred_element_type=jnp.float32
                                ).astype(jnp.bfloat16)
            vv_ref[j] = jnp.dot(hp, wj,
                                preferred_element_type=jnp.float32
                                ).astype(jnp.bfloat16)
        sf_r = jnp.dot(hr, sw, preferred_element_type=jnp.float32)
        sf_p = jnp.dot(hp, sw, preferred_element_type=jnp.float32)
        self_r_ref[0] = sf_r[:, :32]
        self_r_ref[1] = sf_r[:, 32:]
        self_p_ref[0] = sf_p[:, :32]
        self_p_ref[1] = sf_p[:, 32:]

    @pl.when(jnp.logical_and(l == 1, i == 0))
    def _init1():
        hr = jnp.concatenate([h1r_ref[0], h1r_ref[1]], axis=1)
        hp = jnp.concatenate([h1p_ref[0], h1p_ref[1]], axis=1)
        sw = sw1_ref[...]
        for j in range(2):
            wj = w1_ref[j]
            vu_ref[j, :, :16] = jnp.dot(hr, wj,
                                        preferred_element_type=jnp.float32
                                        ).astype(jnp.bfloat16)
            vv_ref[j, :, :16] = jnp.dot(hp, wj,
                                        preferred_element_type=jnp.float32
                                        ).astype(jnp.bfloat16)
        sf_r = jnp.dot(hr, sw, preferred_element_type=jnp.float32)
        sf_p = jnp.dot(hp, sw, preferred_element_type=jnp.float32)
        self_r_ref[0, :, :16] = sf_r[:, :16]
        self_r_ref[1, :, :16] = sf_r[:, 16:]
        self_p_ref[0, :, :16] = sf_p[:, :16]
        self_p_ref[1, :, :16] = sf_p[:, 16:]

    sr = sr_ref[...].astype(jnp.bfloat16)
    sp = sp_ref[...].astype(jnp.bfloat16)

    @pl.when(l == 0)
    def _body0():
        agg_r = jnp.dot(sr, vv_ref[s], preferred_element_type=jnp.float32)
        agg_p = jnp.dot(sp, vu_ref[s], preferred_element_type=jnp.float32)
        h1r_ref[s, rows, :] = jnp.maximum(agg_r + self_r_ref[s, rows, :], 0.0)
        h1p_ref[s, rows, :] = jnp.maximum(agg_p + self_p_ref[s, rows, :], 0.0)

    @pl.when(l == 1)
    def _body1():
        agg_r = jnp.dot(sr, vv_ref[s, :, :16],
                        preferred_element_type=jnp.float32)
        agg_p = jnp.dot(sp, vu_ref[s, :, :16],
                        preferred_element_type=jnp.float32)
        out1r_ref[0] = jnp.maximum(agg_r + self_r_ref[s, rows, :16], 0.0)
        out1p_ref[0] = jnp.maximum(agg_p + self_p_ref[s, rows, :16], 0.0)


def kernel(RNA_supports, protein_supports, RNA_inputs, protein_inputs,
           W0, W1, SW0, SW1):
    sr = RNA_supports.reshape(2 * N, N)
    sp = protein_supports.reshape(2 * N, N)
    sup_spec = pl.BlockSpec((CHUNK, N), lambda l, i: (i, 0))
    full2 = lambda l, i: (0, 0)
    full3 = lambda l, i: (0, 0, 0)
    out_spec = pl.BlockSpec((1, CHUNK, 16), lambda l, i: (i // 16, i % 16, 0))
    o_r, o_p = pl.pallas_call(
        _fused_kernel,
        grid_spec=pltpu.PrefetchScalarGridSpec(
            num_scalar_prefetch=0,
            grid=(2, 2 * N // CHUNK),
            in_specs=[
                sup_spec,
                sup_spec,
                pl.BlockSpec((N, 128), full2),
                pl.BlockSpec((N, 128), full2),
                pl.BlockSpec((2, 128, 32), full3),
                pl.BlockSpec((128, 64), full2),
                pl.BlockSpec((2, 64, 16), full3),
                pl.BlockSpec((64, 32), full2),
            ],
            out_specs=[out_spec, out_spec],
            scratch_shapes=[
                pltpu.VMEM((2, N, 32), jnp.bfloat16),
                pltpu.VMEM((2, N, 32), jnp.bfloat16),
                pltpu.VMEM((2, N, 32), jnp.float32),
                pltpu.VMEM((2, N, 32), jnp.float32),
                pltpu.VMEM((2, N, 32), jnp.float32),
                pltpu.VMEM((2, N, 32), jnp.float32),
            ],
        ),
        out_shape=[
            jax.ShapeDtypeStruct((2, N, 16), jnp.float32),
            jax.ShapeDtypeStruct((2, N, 16), jnp.float32),
        ],
        compiler_params=pltpu.CompilerParams(
            dimension_semantics=("arbitrary", "arbitrary"),
        ),
    )(sr, sp, RNA_inputs, protein_inputs, W0, SW0, W1, SW1)
    out_r = jnp.concatenate([o_r[0], o_r[1]], axis=1)
    out_p = jnp.concatenate([o_p[0], o_p[1]], axis=1)
    return (out_r, out_p)
